# baseline (speedup 1.0000x reference)
import functools

import jax
import jax.numpy as jnp
from jax import lax
from jax.experimental import pallas as pl
from jax.experimental.pallas import tpu as pltpu
from jax.experimental.pallas import tpu_sc as plsc

NUM_EXPERTS_ = 64
TOP_K_ = 8
N_ = 32768 * TOP_K_
NT_ = 16
L_ = 16
U_ = 4
CHUNK_ = N_ // NT_
SSEG_ = CHUNK_ // U_
SEG_ = SSEG_ // L_
ROWS_ = CHUNK_ // 128
UROWS_ = SSEG_ // 128

_mesh = plsc.VectorSubcoreMesh(
    core_axis_name="c", subcore_axis_name="s", num_cores=1, num_subcores=NT_
)
_params = pltpu.CompilerParams(needs_layout_passes=False)


@functools.partial(
    pl.kernel,
    out_type=(
        jax.ShapeDtypeStruct((N_,), jnp.float32),
        jax.ShapeDtypeStruct((N_,), jnp.int32),
        jax.ShapeDtypeStruct((NUM_EXPERTS_,), jnp.int32),
    ),
    mesh=_mesh,
    compiler_params=_params,
    scratch_types=[
        pltpu.VMEM((CHUNK_,), jnp.int32),
        pltpu.VMEM((CHUNK_,), jnp.float32),
        [pltpu.VMEM((NUM_EXPERTS_ * L_,), jnp.int32) for _ in range(U_)],
        [pltpu.VMEM((NUM_EXPERTS_ * L_,), jnp.int32) for _ in range(U_)],
        pltpu.VMEM((NUM_EXPERTS_,), jnp.int32),
        pltpu.VMEM((NT_ * NUM_EXPERTS_,), jnp.int32),
        pltpu.VMEM((NUM_EXPERTS_ + L_,), jnp.int32),
        pltpu.VMEM((ROWS_, 128), jnp.int32),
        pltpu.VMEM((ROWS_, 128), jnp.float32),
        pltpu.VMEM((ROWS_, 128), jnp.int32),
        pltpu.VMEM((NUM_EXPERTS_,), jnp.int32),
        pltpu.VMEM_SHARED((N_,), jnp.float32),
        pltpu.VMEM_SHARED((N_,), jnp.int32),
        pltpu.VMEM_SHARED((NT_ * NUM_EXPERTS_,), jnp.int32),
        pltpu.SemaphoreType.DMA,
        pltpu.SemaphoreType.DMA,
        pltpu.SemaphoreType.DMA,
    ],
)
def _sort_kernel(
    score_hbm,
    idx_hbm,
    score_out_hbm,
    tok_out_hbm,
    hist_out_hbm,
    idx_v,
    score_v,
    hist_u,
    cnt_u,
    histT_v,
    grid_v,
    base_v,
    pos_st,
    score_st,
    tok_st,
    hist_out_v,
    sh_score,
    sh_tok,
    sh_grid,
    sem_in,
    sem0,
    sem1,
):
    t = lax.axis_index("s")
    lane = lax.iota(jnp.int32, L_)
    zeros = jnp.zeros((L_,), jnp.int32)
    ones = jnp.ones((L_,), jnp.int32)

    cp_sc = pltpu.async_copy(
        score_hbm.at[pl.ds(t * CHUNK_, CHUNK_)], score_v, sem_in
    )
    pltpu.sync_copy(idx_hbm.at[pl.ds(t * CHUNK_, CHUNK_)], idx_v)

    def zero_body(i, _):
        for u in range(U_):
            hist_u[u][pl.ds(i * L_, L_)] = zeros
        return 0

    lax.fori_loop(0, NUM_EXPERTS_, zero_body, 0)

    def hist_body(j, _):
        for u in range(U_):
            g = plsc.load_gather(idx_v, [u * SSEG_ + lane * SEG_ + j])
            plsc.addupdate_scatter(hist_u[u], [g * L_ + lane], ones)
        return 0

    lax.fori_loop(0, SEG_, hist_body, 0)

    for grp in range(NUM_EXPERTS_ // L_):
        eids = lane + grp * L_
        acc = jnp.zeros((L_,), jnp.int32)
        for u in range(U_):
            for l in range(L_):
                acc = acc + plsc.load_gather(hist_u[u], [eids * L_ + l])
        histT_v[pl.ds(grp * L_, L_)] = acc

    pltpu.sync_copy(histT_v, sh_grid.at[pl.ds(t * NUM_EXPERTS_, NUM_EXPERTS_)])
    plsc.subcore_barrier()
    pltpu.sync_copy(sh_grid, grid_v)

    tots = []
    pres = []
    for grp in range(NUM_EXPERTS_ // L_):

        def red_body(v, carry):
            tot, pre = carry
            row = grid_v[pl.ds(v * NUM_EXPERTS_ + grp * L_, L_)]
            sel = (v < t).astype(jnp.int32)
            return tot + row, pre + row * sel

        tot, pre = lax.fori_loop(
            0, NT_, red_body, (jnp.zeros((L_,), jnp.int32), jnp.zeros((L_,), jnp.int32))
        )
        tots.append(tot)
        pres.append(pre)

    carry = jnp.int32(0)
    for grp in range(NUM_EXPERTS_ // L_):
        inc = plsc.cumsum(tots[grp])
        excl = inc - tots[grp] + carry
        base_v[pl.ds(L_ + grp * L_, L_)] = excl + pres[grp]
        carry = carry + jnp.sum(tots[grp])

    @pl.when(t == 0)
    def _():
        for grp in range(NUM_EXPERTS_ // L_):
            hist_out_v[pl.ds(grp * L_, L_)] = tots[grp]
        pltpu.sync_copy(hist_out_v, hist_out_hbm)

    for e in range(NUM_EXPERTS_):
        bvec = plsc.load_gather(base_v, [jnp.full((L_,), L_ + e, jnp.int32)])
        spre = jnp.int32(0)
        for u in range(U_):
            row = hist_u[u][pl.ds(e * L_, L_)]
            lexcl = plsc.cumsum(row) - row
            cnt_u[u][pl.ds(e * L_, L_)] = bvec + lexcl + spre
            spre = spre + jnp.sum(row)

    cp_sc.wait()

    def row_body(k, _):
        for m in range(8):
            j = k * 8 + m
            for u in range(U_):
                src = u * SSEG_ + lane * SEG_ + j
                g = plsc.load_gather(idx_v, [src])
                slot = g * L_ + lane
                p = plsc.load_gather(cnt_u[u], [slot])
                plsc.store_scatter(cnt_u[u], [slot], p + 1)
                sc = plsc.load_gather(score_v, [src])
                tok = lax.shift_right_logical(t * CHUNK_ + src, 3)
                r = u * UROWS_ + k
                pos_st[r, pl.ds(m * L_, L_)] = p
                score_st[r, pl.ds(m * L_, L_)] = sc
                tok_st[r, pl.ds(m * L_, L_)] = tok

        for u in range(U_):
            r = u * UROWS_ + k
            pltpu.async_copy(score_st.at[r], sh_score.at[pos_st.at[r]], sem0)
            pltpu.async_copy(tok_st.at[r], sh_tok.at[pos_st.at[r]], sem1)
        return 0

    lax.fori_loop(0, UROWS_, row_body, 0)

    def drain_body(r, _):
        pltpu.make_async_copy(score_st.at[r], sh_score.at[pos_st.at[r]], sem0).wait()
        pltpu.make_async_copy(tok_st.at[r], sh_tok.at[pos_st.at[r]], sem1).wait()
        return 0

    lax.fori_loop(0, ROWS_, drain_body, 0)
    plsc.subcore_barrier()

    pltpu.sync_copy(
        sh_score.at[pl.ds(t * CHUNK_, CHUNK_)],
        score_out_hbm.at[pl.ds(t * CHUNK_, CHUNK_)],
    )
    pltpu.sync_copy(
        sh_tok.at[pl.ds(t * CHUNK_, CHUNK_)],
        tok_out_hbm.at[pl.ds(t * CHUNK_, CHUNK_)],
    )


def kernel(top_scores, selected_experts_indices):
    scores_flat = top_scores.reshape(-1)
    idx_flat = selected_experts_indices.reshape(-1).astype(jnp.int32)
    return _sort_kernel(scores_flat, idx_flat)
